# baseline (speedup 1.0000x reference)
import functools

import jax
import jax.numpy as jnp
from jax import lax
from jax.experimental import pallas as pl
from jax.experimental.pallas import tpu as pltpu
from jax.experimental.pallas import tpu_sc as plsc

P_GEN = 0.2
P_MEM = 0.1
BASE_KEY_SEED = 12345

_B = 256
_C = 512
_FIXED = int(P_GEN * _C)
_KEEP_THRESH = 838861
_EX_PER_WORKER = 8

_ROTATIONS = ((13, 15, 26, 6), (17, 29, 16, 24))


def _threefry2x32(k1, k2, x1, x2):
    ks0 = k1
    ks1 = k2
    ks2 = k1 ^ k2 ^ jnp.uint32(0x1BD11BDA)
    ks = (ks0, ks1, ks2)
    a = x1 + ks0
    b = x2 + ks1
    for i in range(5):
        for r in _ROTATIONS[i % 2]:
            a = a + b
            b = (b << jnp.uint32(r)) | (b >> jnp.uint32(32 - r))
            b = a ^ b
        a = a + ks[(i + 1) % 3]
        b = b + ks[(i + 2) % 3] + jnp.uint32(i + 1)
    return a, b


def _sc_mask_body(idx_hbm, mask_hbm, idx_v, buf_v):
    cid = lax.axis_index("c")
    sid = lax.axis_index("s")
    wid = sid * 2 + cid
    base = wid * _EX_PER_WORKER
    pltpu.sync_copy(idx_hbm.at[pl.ds(base, _EX_PER_WORKER)],
                    idx_v.at[pl.ds(0, _EX_PER_WORKER)])
    idx_all = idx_v[...].astype(jnp.uint32)
    lane = lax.iota(jnp.int32, 16)
    zero = jnp.zeros((16,), jnp.uint32)

    for e in range(_EX_PER_WORKER):
        idx_e = jax.lax.squeeze(jax.lax.slice(idx_all, (e,), (e + 1,)), (0,))
        fk1, fk2 = _threefry2x32(
            jnp.uint32(0), jnp.uint32(BASE_KEY_SEED), jnp.uint32(0), idx_e
        )
        fk1v = jnp.full((16,), fk1)
        fk2v = jnp.full((16,), fk2)

        def v_body(v, carry, fk1v=fk1v, fk2v=fk2v, e=e):
            c = v * 16 + lane
            j = (c - _FIXED).astype(jnp.uint32)
            o1, o2 = _threefry2x32(fk1v, fk2v, zero, j)
            bits = (o1 ^ o2) >> jnp.uint32(9)
            keep_u = (bits - jnp.uint32(_KEEP_THRESH)) >> jnp.uint32(31)
            fixed_u = (c - _FIXED).astype(jnp.uint32) >> jnp.uint32(31)
            val = (keep_u | fixed_u).astype(jnp.int32).astype(jnp.float32)
            buf_v[e, pl.ds(v * 16, 16)] = val
            return carry

        lax.fori_loop(0, _C // 16, v_body, 0)

    pltpu.sync_copy(buf_v, mask_hbm.at[pl.ds(base, _EX_PER_WORKER)])


_sc_mask = functools.partial(
    pl.kernel,
    out_type=jax.ShapeDtypeStruct((_B, _C), jnp.float32),
    mesh=plsc.VectorSubcoreMesh(core_axis_name="c", subcore_axis_name="s"),
    scratch_types=[
        pltpu.VMEM((16,), jnp.int32),
        pltpu.VMEM((_EX_PER_WORKER, _C), jnp.float32),
    ],
)(_sc_mask_body)


def _tc_mul_kernel(m_ref, x_ref, o_ref):
    o_ref[...] = x_ref[...] * m_ref[...][None, :, :]


@jax.jit
def kernel(X, indices):
    B, C, H, W = X.shape
    hw = H * W
    xt = jnp.transpose(X, (2, 3, 0, 1)).reshape(hw, B, C)

    mask = _sc_mask(indices.astype(jnp.int32))

    BB = 32
    out = pl.pallas_call(
        _tc_mul_kernel,
        grid=(B // BB,),
        in_specs=[
            pl.BlockSpec((BB, C), lambda s: (s, 0)),
            pl.BlockSpec((hw, BB, C), lambda s: (0, s, 0)),
        ],
        out_specs=pl.BlockSpec((hw, BB, C), lambda s: (0, s, 0)),
        out_shape=jax.ShapeDtypeStruct((hw, B, C), X.dtype),
    )(mask, xt)
    return jnp.transpose(out.reshape(H, W, B, C), (2, 3, 0, 1))


# speedup vs baseline: 1.4357x; 1.4357x over previous
import functools

import jax
import jax.numpy as jnp
from jax.experimental import pallas as pl
from jax.experimental.pallas import tpu as pltpu

P_GEN = 0.2
P_MEM = 0.1
BASE_KEY_SEED = 12345

_ROTATIONS = ((13, 15, 26, 6), (17, 29, 16, 24))


def _threefry2x32(k1, k2, x1, x2):
    ks0 = k1
    ks1 = k2
    ks2 = k1 ^ k2 ^ jnp.uint32(0x1BD11BDA)
    ks = (ks0, ks1, ks2)
    a = x1 + ks0
    b = x2 + ks1
    for i in range(5):
        for r in _ROTATIONS[i % 2]:
            a = a + b
            b = (b << jnp.uint32(r)) | (b >> jnp.uint32(32 - r))
            b = a ^ b
        a = a + ks[(i + 1) % 3]
        b = b + ks[(i + 2) % 3] + jnp.uint32(i + 1)
    return a, b


def _mask_table(idx_u32, n_channels, fixed_channels):
    bsz = idx_u32.shape[0]
    zero = jnp.zeros_like(idx_u32)
    fk1, fk2 = _threefry2x32(
        jnp.uint32(0), jnp.uint32(BASE_KEY_SEED), zero, idx_u32
    )
    c = jax.lax.broadcasted_iota(jnp.int32, (bsz, n_channels), 1)
    j = (c - fixed_channels).astype(jnp.uint32)
    o1, o2 = _threefry2x32(fk1, fk2, jnp.zeros_like(j), j)
    bits = o1 ^ o2
    keep = ((bits >> jnp.uint32(9)) < jnp.uint32(838861)).astype(jnp.float32)
    return jnp.where(c < fixed_channels, jnp.float32(1.0), keep)


def _tied_dropout_kernel(idx_ref, x_ref, o_ref, *, fixed_channels):
    n_channels = x_ref.shape[2]
    mask = _mask_table(
        idx_ref[...].astype(jnp.uint32), n_channels, fixed_channels
    )
    o_ref[...] = x_ref[...] * mask[None, :, :]


@jax.jit
def kernel(X, indices):
    B, C, H, W = X.shape
    fixed_channels = int(P_GEN * C)
    hw = H * W
    xt = jnp.transpose(X, (2, 3, 0, 1)).reshape(hw, B, C)
    idx2 = indices.astype(jnp.int32).reshape(B, 1)

    BB = 32
    out = pl.pallas_call(
        functools.partial(_tied_dropout_kernel, fixed_channels=fixed_channels),
        grid=(B // BB,),
        in_specs=[
            pl.BlockSpec((BB, 1), lambda s: (s, 0)),
            pl.BlockSpec((hw, BB, C), lambda s: (0, s, 0)),
        ],
        out_specs=pl.BlockSpec((hw, BB, C), lambda s: (0, s, 0)),
        out_shape=jax.ShapeDtypeStruct((hw, B, C), X.dtype),
    )(idx2, xt)
    return jnp.transpose(out.reshape(H, W, B, C), (2, 3, 0, 1))
